# baseline (speedup 1.0000x reference)
import jax
import jax.numpy as jnp
from jax import lax
from jax.experimental import pallas as pl
from jax.experimental.pallas import tpu as pltpu
from jax.experimental.pallas import tpu_sc as plsc

N_NODES = 10000
D = 128
N_EDGES = 320000

NC = 2
NS = 16
NW = NC * NS
EPW = N_EDGES // NW
K = 50
NCHUNK = EPW // K
BCH = 10
NBLK = NCHUNK // BCH
NDEPTH = 5
NQUAD = BCH // NDEPTH
NPAD = 10240
STRIPE = NPAD // NS
ZROWS = 4
ZCOPIES = STRIPE // ZROWS

_MESH = plsc.VectorSubcoreMesh(
    core_axis_name="c", subcore_axis_name="s", num_cores=NC, num_subcores=NS
)


def _spmm_body(table_hbm, src_hbm, dst_hbm, out_hbm,
               src_v, dst_v, r0, r1, r2, r3, r4, zbuf_v, acc_sh,
               gs0, gs1, gs2, gs3, gs4, ss0, ss1, ss2, ss3, ss4, ixs, zsem):
    rs = (r0, r1, r2, r3, r4)
    gss = (gs0, gs1, gs2, gs3, gs4)
    sss = (ss0, ss1, ss2, ss3, ss4)
    c = lax.axis_index("c")
    s = lax.axis_index("s")
    wid = c * NS + s

    ix_a = pltpu.async_copy(src_hbm.at[wid, 0], src_v.at[0], ixs)
    ix_b = pltpu.async_copy(dst_hbm.at[wid, 0], dst_v.at[0], ixs)

    def _zb(i, carry):
        r = i // (D // 16)
        col = (i % (D // 16)) * 16
        zbuf_v[r, pl.ds(col, 16)] = jnp.zeros((16,), jnp.float32)
        return carry
    lax.fori_loop(0, ZROWS * (D // 16), _zb, 0)

    def _zfire(i, carry):
        pltpu.async_copy(zbuf_v, acc_sh.at[pl.ds(s * STRIPE + i * ZROWS, ZROWS)],
                         zsem)
        return carry
    lax.fori_loop(0, ZCOPIES, _zfire, 0)
    ix_a.wait()
    ix_b.wait()

    def _zdrain(i, carry):
        pltpu.make_async_copy(zbuf_v, acc_sh.at[pl.ds(s * STRIPE, ZROWS)],
                              zsem).wait()
        return carry
    lax.fori_loop(0, ZCOPIES, _zdrain, 0)
    plsc.subcore_barrier()

    for blk in range(NBLK):
        cur = blk % 2
        srcb = src_v.at[cur]
        dstb = dst_v.at[cur]
        if blk > 0:
            pltpu.make_async_copy(src_hbm.at[wid, blk], srcb, ixs).wait()
            pltpu.make_async_copy(dst_hbm.at[wid, blk], dstb, ixs).wait()
        if blk + 1 < NBLK:
            nxt = (blk + 1) % 2
            pltpu.async_copy(src_hbm.at[wid, blk + 1], src_v.at[nxt], ixs)
            pltpu.async_copy(dst_hbm.at[wid, blk + 1], dst_v.at[nxt], ixs)

        for k in range(NDEPTH):
            pltpu.async_copy(table_hbm.at[srcb.at[k]], rs[k], gss[k])

        def _quad(q, carry):
            base = NDEPTH * q
            for k in range(NDEPTH):
                i = base + k
                pltpu.make_async_copy(table_hbm.at[srcb.at[i]],
                                      rs[k], gss[k]).wait()
                pltpu.async_copy(rs[k], acc_sh.at[dstb.at[i]], sss[k],
                                 add=True)
            for k in range(NDEPTH):
                i = base + k
                pltpu.make_async_copy(rs[k], acc_sh.at[dstb.at[i]],
                                      sss[k]).wait()
                pltpu.async_copy(table_hbm.at[srcb.at[i + NDEPTH]],
                                 rs[k], gss[k])
            return carry
        lax.fori_loop(0, NQUAD - 1, _quad, 0)

        base = NDEPTH * (NQUAD - 1)
        for k in range(NDEPTH):
            i = base + k
            pltpu.make_async_copy(table_hbm.at[srcb.at[i]],
                                  rs[k], gss[k]).wait()
            pltpu.async_copy(rs[k], acc_sh.at[dstb.at[i]], sss[k], add=True)
        for k in range(NDEPTH):
            i = base + k
            pltpu.make_async_copy(rs[k], acc_sh.at[dstb.at[i]], sss[k]).wait()

    plsc.subcore_barrier()

    pltpu.sync_copy(acc_sh.at[pl.ds(s * STRIPE, STRIPE)],
                    out_hbm.at[c, pl.ds(s * STRIPE, STRIPE)])


def _spmm_partials(table, src4, dst4):
    return pl.kernel(
        _spmm_body,
        out_type=jax.ShapeDtypeStruct((NC, NPAD, D), jnp.float32),
        mesh=_MESH,
        scratch_types=[
            pltpu.VMEM((2, BCH, K), jnp.int32),
            pltpu.VMEM((2, BCH, K), jnp.int32),
            pltpu.VMEM((K, D), jnp.float32),
            pltpu.VMEM((K, D), jnp.float32),
            pltpu.VMEM((K, D), jnp.float32),
            pltpu.VMEM((K, D), jnp.float32),
            pltpu.VMEM((K, D), jnp.float32),
            pltpu.VMEM((ZROWS, D), jnp.float32),
            pltpu.VMEM_SHARED((NPAD, D), jnp.float32),
            pltpu.SemaphoreType.DMA,
            pltpu.SemaphoreType.DMA,
            pltpu.SemaphoreType.DMA,
            pltpu.SemaphoreType.DMA,
            pltpu.SemaphoreType.DMA,
            pltpu.SemaphoreType.DMA,
            pltpu.SemaphoreType.DMA,
            pltpu.SemaphoreType.DMA,
            pltpu.SemaphoreType.DMA,
            pltpu.SemaphoreType.DMA,
            pltpu.SemaphoreType.DMA,
            pltpu.SemaphoreType.DMA,
        ],
    )(table, src4, dst4)


def _mm_body(x_ref, w_ref, o_ref):
    o_ref[...] = jnp.dot(x_ref[...], w_ref[...],
                         preferred_element_type=jnp.float32)


def _matmul(x, w):
    m = x.shape[0]
    bm = 2000
    return pl.pallas_call(
        _mm_body,
        grid=(m // bm,),
        in_specs=[
            pl.BlockSpec((bm, D), lambda i: (i, 0)),
            pl.BlockSpec((D, D), lambda i: (0, 0)),
        ],
        out_specs=pl.BlockSpec((bm, D), lambda i: (i, 0)),
        out_shape=jax.ShapeDtypeStruct((m, D), jnp.float32),
    )(x, w)


def _combine_body_relu(p_ref, o_ref):
    o_ref[...] = jnp.maximum(p_ref[0] + p_ref[1], 0.0)


def _combine_body_plain(p_ref, o_ref):
    o_ref[...] = p_ref[0] + p_ref[1]


def _combine(partials, relu, rows, bm):
    body = _combine_body_relu if relu else _combine_body_plain
    return pl.pallas_call(
        body,
        grid=(rows // bm,),
        in_specs=[pl.BlockSpec((NC, bm, D), lambda i: (0, i, 0))],
        out_specs=pl.BlockSpec((bm, D), lambda i: (i, 0)),
        out_shape=jax.ShapeDtypeStruct((rows, D), jnp.float32),
    )(partials)


@jax.jit
def kernel(x, edge_index, W1):
    src = edge_index[0].astype(jnp.int32).reshape(NW, NBLK, BCH, K)
    dst = edge_index[1].astype(jnp.int32).reshape(NW, NBLK, BCH, K)
    support = _matmul(x, W1)
    p1 = _spmm_partials(support, src, dst)
    h1 = _combine(p1, relu=True, rows=NPAD, bm=2048)
    p2 = _spmm_partials(h1, src, dst)
    return _combine(p2, relu=False, rows=N_NODES, bm=2000)


# speedup vs baseline: 1.0113x; 1.0113x over previous
import jax
import jax.numpy as jnp
from jax import lax
from jax.experimental import pallas as pl
from jax.experimental.pallas import tpu as pltpu
from jax.experimental.pallas import tpu_sc as plsc

N_NODES = 10000
D = 128
N_EDGES = 320000

NC = 2
NS = 16
NW = NC * NS
EPW = N_EDGES // NW
K = 50
NCHUNK = EPW // K
BCH = 20
NBLK = NCHUNK // BCH
NDEPTH = 4
NQUAD = BCH // NDEPTH
NPAD = 10240
STRIPE = NPAD // NS
ZROWS = 16
ZCOPIES = STRIPE // ZROWS

_MESH = plsc.VectorSubcoreMesh(
    core_axis_name="c", subcore_axis_name="s", num_cores=NC, num_subcores=NS
)


def _spmm_body(table_hbm, src_hbm, dst_hbm, out_hbm,
               src_v, dst_v, r0, r1, r2, r3, zbuf_v, acc_sh,
               gs0, gs1, gs2, gs3, ss0, ss1, ss2, ss3, ixs, zsem):
    rs = (r0, r1, r2, r3)
    gss = (gs0, gs1, gs2, gs3)
    sss = (ss0, ss1, ss2, ss3)
    c = lax.axis_index("c")
    s = lax.axis_index("s")
    wid = c * NS + s

    ix_a = pltpu.async_copy(src_hbm.at[wid, 0], src_v.at[0], ixs)
    ix_b = pltpu.async_copy(dst_hbm.at[wid, 0], dst_v.at[0], ixs)

    def _zb(i, carry):
        r = i // (D // 16)
        col = (i % (D // 16)) * 16
        zbuf_v[r, pl.ds(col, 16)] = jnp.zeros((16,), jnp.float32)
        return carry
    lax.fori_loop(0, ZROWS * (D // 16), _zb, 0)

    def _zfire(i, carry):
        pltpu.async_copy(zbuf_v, acc_sh.at[pl.ds(s * STRIPE + i * ZROWS, ZROWS)],
                         zsem)
        return carry
    lax.fori_loop(0, ZCOPIES, _zfire, 0)
    ix_a.wait()
    ix_b.wait()

    def _zdrain(i, carry):
        pltpu.make_async_copy(zbuf_v, acc_sh.at[pl.ds(s * STRIPE, ZROWS)],
                              zsem).wait()
        return carry
    lax.fori_loop(0, ZCOPIES, _zdrain, 0)
    plsc.subcore_barrier()

    for blk in range(NBLK):
        cur = blk % 2
        srcb = src_v.at[cur]
        dstb = dst_v.at[cur]
        if blk > 0:
            pltpu.make_async_copy(src_hbm.at[wid, blk], srcb, ixs).wait()
            pltpu.make_async_copy(dst_hbm.at[wid, blk], dstb, ixs).wait()
        if blk + 1 < NBLK:
            nxt = (blk + 1) % 2
            pltpu.async_copy(src_hbm.at[wid, blk + 1], src_v.at[nxt], ixs)
            pltpu.async_copy(dst_hbm.at[wid, blk + 1], dst_v.at[nxt], ixs)

        for k in range(NDEPTH):
            pltpu.async_copy(table_hbm.at[srcb.at[k]], rs[k], gss[k])

        def _quad(q, carry):
            base = NDEPTH * q
            for k in range(NDEPTH):
                i = base + k
                pltpu.make_async_copy(table_hbm.at[srcb.at[i]],
                                      rs[k], gss[k]).wait()
                pltpu.async_copy(rs[k], acc_sh.at[dstb.at[i]], sss[k],
                                 add=True)
            for k in range(NDEPTH):
                i = base + k
                pltpu.make_async_copy(rs[k], acc_sh.at[dstb.at[i]],
                                      sss[k]).wait()
                pltpu.async_copy(table_hbm.at[srcb.at[i + NDEPTH]],
                                 rs[k], gss[k])
            return carry
        lax.fori_loop(0, NQUAD - 1, _quad, 0)

        base = NDEPTH * (NQUAD - 1)
        for k in range(NDEPTH):
            i = base + k
            pltpu.make_async_copy(table_hbm.at[srcb.at[i]],
                                  rs[k], gss[k]).wait()
            pltpu.async_copy(rs[k], acc_sh.at[dstb.at[i]], sss[k], add=True)
        for k in range(NDEPTH):
            i = base + k
            pltpu.make_async_copy(rs[k], acc_sh.at[dstb.at[i]], sss[k]).wait()

    plsc.subcore_barrier()

    pltpu.sync_copy(acc_sh.at[pl.ds(s * STRIPE, STRIPE)],
                    out_hbm.at[c, pl.ds(s * STRIPE, STRIPE)])


def _spmm_partials(table, src4, dst4):
    return pl.kernel(
        _spmm_body,
        out_type=jax.ShapeDtypeStruct((NC, NPAD, D), jnp.float32),
        mesh=_MESH,
        scratch_types=[
            pltpu.VMEM((2, BCH, K), jnp.int32),
            pltpu.VMEM((2, BCH, K), jnp.int32),
            pltpu.VMEM((K, D), jnp.float32),
            pltpu.VMEM((K, D), jnp.float32),
            pltpu.VMEM((K, D), jnp.float32),
            pltpu.VMEM((K, D), jnp.float32),
            pltpu.VMEM((ZROWS, D), jnp.float32),
            pltpu.VMEM_SHARED((NPAD, D), jnp.float32),
            pltpu.SemaphoreType.DMA,
            pltpu.SemaphoreType.DMA,
            pltpu.SemaphoreType.DMA,
            pltpu.SemaphoreType.DMA,
            pltpu.SemaphoreType.DMA,
            pltpu.SemaphoreType.DMA,
            pltpu.SemaphoreType.DMA,
            pltpu.SemaphoreType.DMA,
            pltpu.SemaphoreType.DMA,
            pltpu.SemaphoreType.DMA,
        ],
    )(table, src4, dst4)


def _mm_body(x_ref, w_ref, o_ref):
    o_ref[...] = jnp.dot(x_ref[...], w_ref[...],
                         preferred_element_type=jnp.float32)


def _matmul(x, w):
    m = x.shape[0]
    bm = 2000
    return pl.pallas_call(
        _mm_body,
        grid=(m // bm,),
        in_specs=[
            pl.BlockSpec((bm, D), lambda i: (i, 0)),
            pl.BlockSpec((D, D), lambda i: (0, 0)),
        ],
        out_specs=pl.BlockSpec((bm, D), lambda i: (i, 0)),
        out_shape=jax.ShapeDtypeStruct((m, D), jnp.float32),
    )(x, w)


def _combine_body_relu(p_ref, o_ref):
    o_ref[...] = jnp.maximum(p_ref[0] + p_ref[1], 0.0)


def _combine_body_plain(p_ref, o_ref):
    o_ref[...] = p_ref[0] + p_ref[1]


def _combine(partials, relu, rows, bm):
    body = _combine_body_relu if relu else _combine_body_plain
    return pl.pallas_call(
        body,
        grid=(rows // bm,),
        in_specs=[pl.BlockSpec((NC, bm, D), lambda i: (0, i, 0))],
        out_specs=pl.BlockSpec((bm, D), lambda i: (i, 0)),
        out_shape=jax.ShapeDtypeStruct((rows, D), jnp.float32),
    )(partials)


@jax.jit
def kernel(x, edge_index, W1):
    src = edge_index[0].astype(jnp.int32).reshape(NW, NBLK, BCH, K)
    dst = edge_index[1].astype(jnp.int32).reshape(NW, NBLK, BCH, K)
    support = _matmul(x, W1)
    p1 = _spmm_partials(support, src, dst)
    h1 = _combine(p1, relu=True, rows=NPAD, bm=2048)
    p2 = _spmm_partials(h1, src, dst)
    return _combine(p2, relu=False, rows=N_NODES, bm=2000)


# speedup vs baseline: 1.0340x; 1.0224x over previous
import jax
import jax.numpy as jnp
from jax import lax
from jax.experimental import pallas as pl
from jax.experimental.pallas import tpu as pltpu
from jax.experimental.pallas import tpu_sc as plsc

N_NODES = 10000
D = 128
N_EDGES = 320000

NC = 2
NS = 16
NW = NC * NS
EPW = N_EDGES // NW
K = 50
NCHUNK = EPW // K
BCH = 20
NBLK = NCHUNK // BCH
NDEPTH = 4
NQUAD = BCH // NDEPTH
NPAD = 10240
STRIPE = NPAD // NS
ZROWS = 16
ZCOPIES = STRIPE // ZROWS

_MESH = plsc.VectorSubcoreMesh(
    core_axis_name="c", subcore_axis_name="s", num_cores=NC, num_subcores=NS
)


def _spmm_body(table_hbm, src_hbm, dst_hbm, out_hbm,
               src_v, dst_v, r0, r1, r2, r3, zbuf_v, acc_sh,
               gs0, gs1, gs2, gs3, ss0, ss1, ss2, ss3, ixs, zsem):
    rs = (r0, r1, r2, r3)
    gss = (gs0, gs1, gs2, gs3)
    sss = (ss0, ss1, ss2, ss3)
    c = lax.axis_index("c")
    s = lax.axis_index("s")
    wid = c * NS + s

    ix_a = pltpu.async_copy(src_hbm.at[wid, 0], src_v.at[0], ixs)
    ix_b = pltpu.async_copy(dst_hbm.at[wid, 0], dst_v.at[0], ixs)

    def _zb(i, carry):
        r = i // (D // 16)
        col = (i % (D // 16)) * 16
        zbuf_v[r, pl.ds(col, 16)] = jnp.zeros((16,), jnp.float32)
        return carry
    lax.fori_loop(0, ZROWS * (D // 16), _zb, 0)

    def _zfire(i, carry):
        pltpu.async_copy(zbuf_v, acc_sh.at[pl.ds(s * STRIPE + i * ZROWS, ZROWS)],
                         zsem)
        return carry
    lax.fori_loop(0, ZCOPIES, _zfire, 0)
    ix_a.wait()
    ix_b.wait()

    def _zdrain(i, carry):
        pltpu.make_async_copy(zbuf_v, acc_sh.at[pl.ds(s * STRIPE, ZROWS)],
                              zsem).wait()
        return carry
    lax.fori_loop(0, ZCOPIES, _zdrain, 0)
    plsc.subcore_barrier()

    for blk in range(NBLK):
        cur = blk % 2
        srcb = src_v.at[cur]
        dstb = dst_v.at[cur]
        if blk > 0:
            pltpu.make_async_copy(src_hbm.at[wid, blk], srcb, ixs).wait()
            pltpu.make_async_copy(dst_hbm.at[wid, blk], dstb, ixs).wait()
        if blk + 1 < NBLK:
            nxt = (blk + 1) % 2
            pltpu.async_copy(src_hbm.at[wid, blk + 1], src_v.at[nxt], ixs)
            pltpu.async_copy(dst_hbm.at[wid, blk + 1], dst_v.at[nxt], ixs)

        for k in range(NDEPTH):
            pltpu.async_copy(table_hbm.at[srcb.at[k]], rs[k], gss[k])

        def _quad(q, carry):
            base = NDEPTH * q
            for k in range(NDEPTH):
                i = base + k
                pltpu.make_async_copy(table_hbm.at[srcb.at[i]],
                                      rs[k], gss[k]).wait()
                pltpu.async_copy(rs[k], acc_sh.at[dstb.at[i]], sss[k],
                                 add=True)
            for k in range(NDEPTH):
                i = base + k
                pltpu.make_async_copy(rs[k], acc_sh.at[dstb.at[i]],
                                      sss[k]).wait()
                pltpu.async_copy(table_hbm.at[srcb.at[i + NDEPTH]],
                                 rs[k], gss[k])
            return carry
        lax.fori_loop(0, NQUAD - 1, _quad, 0)

        base = NDEPTH * (NQUAD - 1)
        for k in range(NDEPTH):
            i = base + k
            pltpu.make_async_copy(table_hbm.at[srcb.at[i]],
                                  rs[k], gss[k]).wait()
            pltpu.async_copy(rs[k], acc_sh.at[dstb.at[i]], sss[k], add=True)
        for k in range(NDEPTH):
            i = base + k
            pltpu.make_async_copy(rs[k], acc_sh.at[dstb.at[i]], sss[k]).wait()

    plsc.subcore_barrier()

    pltpu.sync_copy(acc_sh.at[pl.ds(s * STRIPE, STRIPE)],
                    out_hbm.at[c, pl.ds(s * STRIPE, STRIPE)])


def _spmm_partials(table, src4, dst4):
    return pl.kernel(
        _spmm_body,
        out_type=jax.ShapeDtypeStruct((NC, NPAD, D), jnp.float32),
        mesh=_MESH,
        scratch_types=[
            pltpu.VMEM((2, BCH, K), jnp.int32),
            pltpu.VMEM((2, BCH, K), jnp.int32),
            pltpu.VMEM((K, D), jnp.float32),
            pltpu.VMEM((K, D), jnp.float32),
            pltpu.VMEM((K, D), jnp.float32),
            pltpu.VMEM((K, D), jnp.float32),
            pltpu.VMEM((ZROWS, D), jnp.float32),
            pltpu.VMEM_SHARED((NPAD, D), jnp.float32),
            pltpu.SemaphoreType.DMA,
            pltpu.SemaphoreType.DMA,
            pltpu.SemaphoreType.DMA,
            pltpu.SemaphoreType.DMA,
            pltpu.SemaphoreType.DMA,
            pltpu.SemaphoreType.DMA,
            pltpu.SemaphoreType.DMA,
            pltpu.SemaphoreType.DMA,
            pltpu.SemaphoreType.DMA,
            pltpu.SemaphoreType.DMA,
        ],
    )(table, src4, dst4)


def _mm_relu_body(p_ref, w_ref, o_ref):
    o_ref[...] = jnp.maximum(
        jnp.dot(p_ref[0] + p_ref[1], w_ref[...],
                preferred_element_type=jnp.float32), 0.0)


def _combine_mm_relu(partials, w):
    bm = 2048
    return pl.pallas_call(
        _mm_relu_body,
        grid=(NPAD // bm,),
        in_specs=[
            pl.BlockSpec((NC, bm, D), lambda i: (0, i, 0)),
            pl.BlockSpec((D, D), lambda i: (0, 0)),
        ],
        out_specs=pl.BlockSpec((bm, D), lambda i: (i, 0)),
        out_shape=jax.ShapeDtypeStruct((NPAD, D), jnp.float32),
    )(partials, w)


def _combine_body_plain(p_ref, o_ref):
    o_ref[...] = p_ref[0] + p_ref[1]


def _combine(partials, rows, bm):
    body = _combine_body_plain
    return pl.pallas_call(
        body,
        grid=(rows // bm,),
        in_specs=[pl.BlockSpec((NC, bm, D), lambda i: (0, i, 0))],
        out_specs=pl.BlockSpec((bm, D), lambda i: (i, 0)),
        out_shape=jax.ShapeDtypeStruct((rows, D), jnp.float32),
    )(partials)


@jax.jit
def kernel(x, edge_index, W1):
    src = edge_index[0].astype(jnp.int32).reshape(NW, NBLK, BCH, K)
    dst = edge_index[1].astype(jnp.int32).reshape(NW, NBLK, BCH, K)
    p1 = _spmm_partials(x, src, dst)
    h1 = _combine_mm_relu(p1, W1)
    p2 = _spmm_partials(h1, src, dst)
    return _combine(p2, rows=N_NODES, bm=2000)


# speedup vs baseline: 1.0400x; 1.0058x over previous
import jax
import jax.numpy as jnp
from jax import lax
from jax.experimental import pallas as pl
from jax.experimental.pallas import tpu as pltpu
from jax.experimental.pallas import tpu_sc as plsc

N_NODES = 10000
D = 128
N_EDGES = 320000

NC = 2
NS = 16
NW = NC * NS
EPW = N_EDGES // NW
K = 50
NCHUNK = EPW // K
BCH = 20
NBLK = NCHUNK // BCH
NDEPTH = 4
NQUAD = BCH // NDEPTH
NPAD = 10240
STRIPE = NPAD // NS
ZROWS = 16
ZCOPIES = STRIPE // ZROWS

_MESH = plsc.VectorSubcoreMesh(
    core_axis_name="c", subcore_axis_name="s", num_cores=NC, num_subcores=NS
)


def _spmm_body(table_hbm, src_hbm, dst_hbm, out_hbm,
               src_v, dst_v, r0, r1, r2, r3, zbuf_v, acc_sh,
               gs0, gs1, gs2, gs3, ss0, ss1, ss2, ss3, ixs, zsem):
    rs = (r0, r1, r2, r3)
    gss = (gs0, gs1, gs2, gs3)
    sss = (ss0, ss1, ss2, ss3)
    c = lax.axis_index("c")
    s = lax.axis_index("s")
    wid = c * NS + s

    ix_a = pltpu.async_copy(src_hbm.at[wid, 0], src_v.at[0], ixs)
    ix_b = pltpu.async_copy(dst_hbm.at[wid, 0], dst_v.at[0], ixs)

    def _zb(i, carry):
        r = i // (D // 16)
        col = (i % (D // 16)) * 16
        zbuf_v[r, pl.ds(col, 16)] = jnp.zeros((16,), jnp.float32)
        return carry
    lax.fori_loop(0, ZROWS * (D // 16), _zb, 0)

    def _zfire(i, carry):
        pltpu.async_copy(zbuf_v, acc_sh.at[pl.ds(s * STRIPE + i * ZROWS, ZROWS)],
                         zsem)
        return carry
    lax.fori_loop(0, ZCOPIES, _zfire, 0)
    ix_a.wait()
    ix_b.wait()

    def _zdrain(i, carry):
        pltpu.make_async_copy(zbuf_v, acc_sh.at[pl.ds(s * STRIPE, ZROWS)],
                              zsem).wait()
        return carry
    lax.fori_loop(0, ZCOPIES, _zdrain, 0)
    plsc.subcore_barrier()

    for blk in range(NBLK):
        cur = blk % 2
        srcb = src_v.at[cur]
        dstb = dst_v.at[cur]
        if blk > 0:
            pltpu.make_async_copy(src_hbm.at[wid, blk], srcb, ixs).wait()
            pltpu.make_async_copy(dst_hbm.at[wid, blk], dstb, ixs).wait()
        if blk + 1 < NBLK:
            nxt = (blk + 1) % 2
            pltpu.async_copy(src_hbm.at[wid, blk + 1], src_v.at[nxt], ixs)
            pltpu.async_copy(dst_hbm.at[wid, blk + 1], dst_v.at[nxt], ixs)

        for k in range(NDEPTH):
            pltpu.async_copy(table_hbm.at[srcb.at[k]], rs[k], gss[k])

        def _quad(q, carry):
            base = NDEPTH * q
            for k in range(NDEPTH):
                i = base + k
                pltpu.make_async_copy(table_hbm.at[srcb.at[i]],
                                      rs[k], gss[k]).wait()
                pltpu.async_copy(rs[k], acc_sh.at[dstb.at[i]], sss[k],
                                 add=True)
            for k in range(NDEPTH):
                i = base + k
                pltpu.make_async_copy(rs[k], acc_sh.at[dstb.at[i]],
                                      sss[k]).wait()
                pltpu.async_copy(table_hbm.at[srcb.at[i + NDEPTH]],
                                 rs[k], gss[k])
            return carry
        lax.fori_loop(0, NQUAD - 1, _quad, 0)

        base = NDEPTH * (NQUAD - 1)
        for k in range(NDEPTH):
            i = base + k
            pltpu.make_async_copy(table_hbm.at[srcb.at[i]],
                                  rs[k], gss[k]).wait()
            pltpu.async_copy(rs[k], acc_sh.at[dstb.at[i]], sss[k], add=True)
        for k in range(NDEPTH):
            i = base + k
            pltpu.make_async_copy(rs[k], acc_sh.at[dstb.at[i]], sss[k]).wait()

    plsc.subcore_barrier()

    pltpu.sync_copy(acc_sh.at[pl.ds(s * STRIPE, STRIPE)],
                    out_hbm.at[c, pl.ds(s * STRIPE, STRIPE)])


def _spmm_partials(table, src4, dst4):
    return pl.kernel(
        _spmm_body,
        out_type=jax.ShapeDtypeStruct((NC, NPAD, D), jnp.float32),
        mesh=_MESH,
        scratch_types=[
            pltpu.VMEM((2, BCH, K), jnp.int32),
            pltpu.VMEM((2, BCH, K), jnp.int32),
            pltpu.VMEM((K, D), jnp.float32),
            pltpu.VMEM((K, D), jnp.float32),
            pltpu.VMEM((K, D), jnp.float32),
            pltpu.VMEM((K, D), jnp.float32),
            pltpu.VMEM((ZROWS, D), jnp.float32),
            pltpu.VMEM_SHARED((NPAD, D), jnp.float32),
            pltpu.SemaphoreType.DMA,
            pltpu.SemaphoreType.DMA,
            pltpu.SemaphoreType.DMA,
            pltpu.SemaphoreType.DMA,
            pltpu.SemaphoreType.DMA,
            pltpu.SemaphoreType.DMA,
            pltpu.SemaphoreType.DMA,
            pltpu.SemaphoreType.DMA,
            pltpu.SemaphoreType.DMA,
            pltpu.SemaphoreType.DMA,
        ],
    )(table, src4, dst4)


def _mm_relu_body(p_ref, w_ref, o_ref):
    o_ref[...] = jnp.maximum(
        jnp.dot(p_ref[0] + p_ref[1], w_ref[...],
                preferred_element_type=jnp.float32), 0.0)


def _combine_mm_relu(partials, w):
    bm = NPAD
    return pl.pallas_call(
        _mm_relu_body,
        grid=(NPAD // bm,),
        in_specs=[
            pl.BlockSpec((NC, bm, D), lambda i: (0, i, 0)),
            pl.BlockSpec((D, D), lambda i: (0, 0)),
        ],
        out_specs=pl.BlockSpec((bm, D), lambda i: (i, 0)),
        out_shape=jax.ShapeDtypeStruct((NPAD, D), jnp.float32),
    )(partials, w)


def _combine_body_plain(p_ref, o_ref):
    o_ref[...] = p_ref[0] + p_ref[1]


def _combine(partials, rows, bm):
    body = _combine_body_plain
    return pl.pallas_call(
        body,
        grid=(rows // bm,),
        in_specs=[pl.BlockSpec((NC, bm, D), lambda i: (0, i, 0))],
        out_specs=pl.BlockSpec((bm, D), lambda i: (i, 0)),
        out_shape=jax.ShapeDtypeStruct((rows, D), jnp.float32),
    )(partials)


@jax.jit
def kernel(x, edge_index, W1):
    src = edge_index[0].astype(jnp.int32).reshape(NW, NBLK, BCH, K)
    dst = edge_index[1].astype(jnp.int32).reshape(NW, NBLK, BCH, K)
    p1 = _spmm_partials(x, src, dst)
    h1 = _combine_mm_relu(p1, W1)
    p2 = _spmm_partials(h1, src, dst)
    return _combine(p2, rows=N_NODES, bm=N_NODES)


# speedup vs baseline: 1.0511x; 1.0107x over previous
import jax
import jax.numpy as jnp
from jax import lax
from jax.experimental import pallas as pl
from jax.experimental.pallas import tpu as pltpu
from jax.experimental.pallas import tpu_sc as plsc

N_NODES = 10000
D = 128
N_EDGES = 320000

NC = 2
NS = 16
NW = NC * NS
EPW = N_EDGES // NW
K = 50
NCHUNK = EPW // K
BCH = 20
NBLK = NCHUNK // BCH
NDEPTH = 4
NQUAD = BCH // NDEPTH
NPAD = 10240
STRIPE = NPAD // NS
ZROWS = 16
ZCOPIES = STRIPE // ZROWS

_MESH = plsc.VectorSubcoreMesh(
    core_axis_name="c", subcore_axis_name="s", num_cores=NC, num_subcores=NS
)


def _spmm_body(table_hbm, src_hbm, dst_hbm, out_hbm,
               src_v, dst_v, r0, r1, r2, r3, zbuf_v, acc_sh,
               gs0, gs1, gs2, gs3, ss0, ss1, ss2, ss3, ixs, zsem):
    rs = (r0, r1, r2, r3)
    gss = (gs0, gs1, gs2, gs3)
    sss = (ss0, ss1, ss2, ss3)
    c = lax.axis_index("c")
    s = lax.axis_index("s")
    wid = c * NS + s

    ix_a = pltpu.async_copy(src_hbm.at[wid, 0], src_v.at[0], ixs)
    ix_b = pltpu.async_copy(dst_hbm.at[wid, 0], dst_v.at[0], ixs)

    def _zb(i, carry):
        r = i // (D // 16)
        col = (i % (D // 16)) * 16
        zbuf_v[r, pl.ds(col, 16)] = jnp.zeros((16,), jnp.float32)
        return carry
    lax.fori_loop(0, ZROWS * (D // 16), _zb, 0)

    def _zfire(i, carry):
        pltpu.async_copy(zbuf_v, acc_sh.at[pl.ds(s * STRIPE + i * ZROWS, ZROWS)],
                         zsem)
        return carry
    lax.fori_loop(0, ZCOPIES, _zfire, 0)
    ix_a.wait()
    ix_b.wait()

    for k in range(NDEPTH):
        pltpu.async_copy(table_hbm.at[src_v.at[0].at[k]], rs[k], gss[k])

    def _zdrain(i, carry):
        pltpu.make_async_copy(zbuf_v, acc_sh.at[pl.ds(s * STRIPE, ZROWS)],
                              zsem).wait()
        return carry
    lax.fori_loop(0, ZCOPIES, _zdrain, 0)
    plsc.subcore_barrier()

    for blk in range(NBLK):
        cur = blk % 2
        srcb = src_v.at[cur]
        dstb = dst_v.at[cur]
        if blk > 0:
            pltpu.make_async_copy(src_hbm.at[wid, blk], srcb, ixs).wait()
            pltpu.make_async_copy(dst_hbm.at[wid, blk], dstb, ixs).wait()
        if blk + 1 < NBLK:
            nxt = (blk + 1) % 2
            pltpu.async_copy(src_hbm.at[wid, blk + 1], src_v.at[nxt], ixs)
            pltpu.async_copy(dst_hbm.at[wid, blk + 1], dst_v.at[nxt], ixs)

        if blk > 0:
            for k in range(NDEPTH):
                pltpu.async_copy(table_hbm.at[srcb.at[k]], rs[k], gss[k])

        def _quad(q, carry):
            base = NDEPTH * q
            for k in range(NDEPTH):
                i = base + k
                pltpu.make_async_copy(table_hbm.at[srcb.at[i]],
                                      rs[k], gss[k]).wait()
                pltpu.async_copy(rs[k], acc_sh.at[dstb.at[i]], sss[k],
                                 add=True)
            for k in range(NDEPTH):
                i = base + k
                pltpu.make_async_copy(rs[k], acc_sh.at[dstb.at[i]],
                                      sss[k]).wait()
                pltpu.async_copy(table_hbm.at[srcb.at[i + NDEPTH]],
                                 rs[k], gss[k])
            return carry
        lax.fori_loop(0, NQUAD - 1, _quad, 0)

        base = NDEPTH * (NQUAD - 1)
        for k in range(NDEPTH):
            i = base + k
            pltpu.make_async_copy(table_hbm.at[srcb.at[i]],
                                  rs[k], gss[k]).wait()
            pltpu.async_copy(rs[k], acc_sh.at[dstb.at[i]], sss[k], add=True)
        for k in range(NDEPTH):
            i = base + k
            pltpu.make_async_copy(rs[k], acc_sh.at[dstb.at[i]], sss[k]).wait()

    plsc.subcore_barrier()

    pltpu.sync_copy(acc_sh.at[pl.ds(s * STRIPE, STRIPE)],
                    out_hbm.at[c, pl.ds(s * STRIPE, STRIPE)])


def _spmm_partials(table, src4, dst4):
    return pl.kernel(
        _spmm_body,
        out_type=jax.ShapeDtypeStruct((NC, NPAD, D), jnp.float32),
        mesh=_MESH,
        scratch_types=[
            pltpu.VMEM((2, BCH, K), jnp.int32),
            pltpu.VMEM((2, BCH, K), jnp.int32),
            pltpu.VMEM((K, D), jnp.float32),
            pltpu.VMEM((K, D), jnp.float32),
            pltpu.VMEM((K, D), jnp.float32),
            pltpu.VMEM((K, D), jnp.float32),
            pltpu.VMEM((ZROWS, D), jnp.float32),
            pltpu.VMEM_SHARED((NPAD, D), jnp.float32),
            pltpu.SemaphoreType.DMA,
            pltpu.SemaphoreType.DMA,
            pltpu.SemaphoreType.DMA,
            pltpu.SemaphoreType.DMA,
            pltpu.SemaphoreType.DMA,
            pltpu.SemaphoreType.DMA,
            pltpu.SemaphoreType.DMA,
            pltpu.SemaphoreType.DMA,
            pltpu.SemaphoreType.DMA,
            pltpu.SemaphoreType.DMA,
        ],
    )(table, src4, dst4)


def _mm_relu_body(p_ref, w_ref, o_ref):
    o_ref[...] = jnp.maximum(
        jnp.dot(p_ref[0] + p_ref[1], w_ref[...],
                preferred_element_type=jnp.float32), 0.0)


def _combine_mm_relu(partials, w):
    bm = NPAD
    return pl.pallas_call(
        _mm_relu_body,
        grid=(NPAD // bm,),
        in_specs=[
            pl.BlockSpec((NC, bm, D), lambda i: (0, i, 0)),
            pl.BlockSpec((D, D), lambda i: (0, 0)),
        ],
        out_specs=pl.BlockSpec((bm, D), lambda i: (i, 0)),
        out_shape=jax.ShapeDtypeStruct((NPAD, D), jnp.float32),
    )(partials, w)


def _combine_body_plain(p_ref, o_ref):
    o_ref[...] = p_ref[0] + p_ref[1]


def _combine(partials, rows, bm):
    body = _combine_body_plain
    return pl.pallas_call(
        body,
        grid=(rows // bm,),
        in_specs=[pl.BlockSpec((NC, bm, D), lambda i: (0, i, 0))],
        out_specs=pl.BlockSpec((bm, D), lambda i: (i, 0)),
        out_shape=jax.ShapeDtypeStruct((rows, D), jnp.float32),
    )(partials)


@jax.jit
def kernel(x, edge_index, W1):
    src = edge_index[0].astype(jnp.int32).reshape(NW, NBLK, BCH, K)
    dst = edge_index[1].astype(jnp.int32).reshape(NW, NBLK, BCH, K)
    p1 = _spmm_partials(x, src, dst)
    h1 = _combine_mm_relu(p1, W1)
    p2 = _spmm_partials(h1, src, dst)
    return _combine(p2, rows=N_NODES, bm=N_NODES)
